# baseline (speedup 1.0000x reference)
import functools

import jax
import jax.numpy as jnp
from jax import lax
from jax.experimental import pallas as pl
from jax.experimental.pallas import tpu as pltpu
from jax.experimental.pallas import tpu_sc as plsc

N_FIELDS = 26
VOCAB = 100000
DIM = 64
BATCH = 4096
TOT = BATCH * N_FIELDS
NC, NS = 2, 16
NW = NC * NS
PER_W = TOT // NW
CHUNK = 128
NCH = PER_W // CHUNK
CH_PER_F = BATCH // CHUNK
LANES = 16
TLANE = 2048
TGRID = -(-VOCAB // TLANE)


def _transpose_body(tt_ref, tp_ref):
    eye = jnp.eye(DIM, dtype=jnp.float32)
    blk = lax.dot_general(
        tt_ref[0], eye, (((0,), (0,)), ((), ())),
        preferred_element_type=jnp.float32)
    tp_ref[0] = jnp.concatenate([blk[:TLANE // 2], blk[TLANE // 2:]], axis=1)


def _transpose(tt):
    return pl.pallas_call(
        _transpose_body,
        grid=(N_FIELDS, TGRID),
        in_specs=[pl.BlockSpec((1, DIM, TLANE), lambda i, c: (i, 0, c))],
        out_specs=pl.BlockSpec((1, TLANE // 2, 2 * DIM),
                               lambda i, c: (i, c, 0)),
        out_shape=jax.ShapeDtypeStruct(
            (N_FIELDS, TGRID * TLANE // 2, 2 * DIM), jnp.float32),
        compiler_params=pltpu.CompilerParams(
            dimension_semantics=("arbitrary", "arbitrary")),
    )(tt)


def _embed_body(xflat, tablep, out, idx_v, q_v, h_v, rows_v, ob_v, gsem):
    wid = lax.axis_index("s") * NC + lax.axis_index("c")
    pltpu.sync_copy(xflat.at[wid], idx_v)

    @pl.loop(0, NCH)
    def _chunk(j):
        ch = wid * NCH + j
        fld = ch // CH_PER_F
        b0 = (ch % CH_PER_F) * CHUNK
        off = fld * (TGRID * TLANE // 2)
        for t in range(CHUNK // LANES):
            sl = pl.ds(t * LANES, LANES)
            r = idx_v[j, sl]
            q_v[sl] = (lax.shift_right_logical(r, 11) * (TLANE // 2)
                       + (r & (TLANE // 2 - 1)) + off)
            h_v[sl] = (lax.shift_right_logical(r, 10) & 1) * DIM
        pltpu.async_copy(tablep.at[q_v], rows_v, gsem).wait()
        for g in range(CHUNK // LANES):
            bvec = lax.iota(jnp.int32, LANES) + g * LANES
            hs = h_v[pl.ds(g * LANES, LANES)]

            @pl.loop(0, DIM, unroll=8)
            def _col(c):
                vals = plsc.load_gather(rows_v, [bvec, hs + c])
                ob_v[c, pl.ds(g * LANES, LANES)] = vals
        pltpu.sync_copy(ob_v, out.at[fld, :, pl.ds(b0, CHUNK)])


@functools.partial(
    pl.kernel,
    out_type=jax.ShapeDtypeStruct((N_FIELDS, DIM, BATCH), jnp.float32),
    mesh=plsc.VectorSubcoreMesh(core_axis_name="c", subcore_axis_name="s"),
    compiler_params=pltpu.CompilerParams(needs_layout_passes=False),
    scratch_types=[
        pltpu.VMEM((NCH, CHUNK), jnp.int32),
        pltpu.VMEM((CHUNK,), jnp.int32),
        pltpu.VMEM((CHUNK,), jnp.int32),
        pltpu.VMEM((CHUNK, 2 * DIM), jnp.float32),
        pltpu.VMEM((DIM, CHUNK), jnp.float32),
        pltpu.SemaphoreType.DMA,
    ],
)
def _embed(xflat, tablep, out, idx_v, q_v, h_v, rows_v, ob_v, gsem):
    _embed_body(xflat, tablep, out, idx_v, q_v, h_v, rows_v, ob_v, gsem)


def kernel(x, tables):
    tt = jnp.swapaxes(tables, 1, 2)
    xflat = jnp.swapaxes(x, 0, 1).reshape(NW, NCH, CHUNK)
    tp = _transpose(tt)
    tablep = tp.reshape(N_FIELDS * (TGRID * TLANE // 2), 2 * DIM)
    otv = _embed(xflat, tablep)
    return jnp.transpose(otv, (2, 0, 1))


# speedup vs baseline: 1.5425x; 1.5425x over previous
import functools

import jax
import jax.numpy as jnp
from jax import lax
from jax.experimental import pallas as pl
from jax.experimental.pallas import tpu as pltpu
from jax.experimental.pallas import tpu_sc as plsc

N_FIELDS = 26
VOCAB = 100000
DIM = 64
BATCH = 4096
TOT = BATCH * N_FIELDS
NC, NS = 2, 16
NW = NC * NS
PER_W = TOT // NW
CHUNK = 128
NCH = PER_W // CHUNK
CH_PER_F = BATCH // CHUNK
LANES = 16
TLANE = 8192
TGRID = -(-VOCAB // TLANE)


def _transpose_body(tt_ref, tp_ref):
    blk = jnp.swapaxes(tt_ref[0], 0, 1)
    tp_ref[0] = jnp.concatenate([blk[:TLANE // 2], blk[TLANE // 2:]], axis=1)


def _transpose(tt):
    return pl.pallas_call(
        _transpose_body,
        grid=(N_FIELDS, TGRID),
        in_specs=[pl.BlockSpec((1, DIM, TLANE), lambda i, c: (i, 0, c))],
        out_specs=pl.BlockSpec((1, TLANE // 2, 2 * DIM),
                               lambda i, c: (i, c, 0)),
        out_shape=jax.ShapeDtypeStruct(
            (N_FIELDS, TGRID * TLANE // 2, 2 * DIM), jnp.float32),
        compiler_params=pltpu.CompilerParams(
            dimension_semantics=("arbitrary", "arbitrary")),
    )(tt)


def _embed_body(xflat, tablep, out, idx_v, q_v, h_v, rows_v, ob_v, gsem):
    wid = lax.axis_index("s") * NC + lax.axis_index("c")
    pltpu.sync_copy(xflat.at[wid], idx_v)

    @pl.loop(0, NCH)
    def _chunk(j):
        ch = wid * NCH + j
        fld = ch // CH_PER_F
        b0 = (ch % CH_PER_F) * CHUNK
        off = fld * (TGRID * TLANE // 2)
        for t in range(CHUNK // LANES):
            sl = pl.ds(t * LANES, LANES)
            r = idx_v[j, sl]
            q_v[sl] = (lax.shift_right_logical(r, TLANE.bit_length() - 1)
                       * (TLANE // 2) + (r & (TLANE // 2 - 1)) + off)
            h_v[sl] = (lax.shift_right_logical(r, TLANE.bit_length() - 2)
                       & 1) * DIM
        pltpu.async_copy(tablep.at[q_v], rows_v, gsem).wait()
        for g in range(CHUNK // LANES):
            bvec = lax.iota(jnp.int32, LANES) + g * LANES
            hs = h_v[pl.ds(g * LANES, LANES)]

            @pl.loop(0, DIM, unroll=8)
            def _col(c):
                vals = plsc.load_gather(rows_v, [bvec, hs + c])
                ob_v[c, pl.ds(g * LANES, LANES)] = vals
        pltpu.sync_copy(ob_v, out.at[fld, :, pl.ds(b0, CHUNK)])


@functools.partial(
    pl.kernel,
    out_type=jax.ShapeDtypeStruct((N_FIELDS, DIM, BATCH), jnp.float32),
    mesh=plsc.VectorSubcoreMesh(core_axis_name="c", subcore_axis_name="s"),
    compiler_params=pltpu.CompilerParams(needs_layout_passes=False),
    scratch_types=[
        pltpu.VMEM((NCH, CHUNK), jnp.int32),
        pltpu.VMEM((CHUNK,), jnp.int32),
        pltpu.VMEM((CHUNK,), jnp.int32),
        pltpu.VMEM((CHUNK, 2 * DIM), jnp.float32),
        pltpu.VMEM((DIM, CHUNK), jnp.float32),
        pltpu.SemaphoreType.DMA,
    ],
)
def _embed(xflat, tablep, out, idx_v, q_v, h_v, rows_v, ob_v, gsem):
    _embed_body(xflat, tablep, out, idx_v, q_v, h_v, rows_v, ob_v, gsem)


def kernel(x, tables):
    tt = jnp.swapaxes(tables, 1, 2)
    xflat = jnp.swapaxes(x, 0, 1).reshape(NW, NCH, CHUNK)
    tp = _transpose(tt)
    tablep = tp.reshape(N_FIELDS * (TGRID * TLANE // 2), 2 * DIM)
    otv = _embed(xflat, tablep)
    return jnp.transpose(otv, (2, 0, 1))


# speedup vs baseline: 1.6227x; 1.0519x over previous
import functools

import jax
import jax.numpy as jnp
from jax import lax
from jax.experimental import pallas as pl
from jax.experimental.pallas import tpu as pltpu
from jax.experimental.pallas import tpu_sc as plsc

N_FIELDS = 26
VOCAB = 100000
DIM = 64
BATCH = 4096
TOT = BATCH * N_FIELDS
NC, NS = 2, 16
NW = NC * NS
PER_W = TOT // NW
CHUNK = 128
NCH = PER_W // CHUNK
CH_PER_F = BATCH // CHUNK
LANES = 16
TLANE = 8192
TGRID = -(-VOCAB // TLANE)


def _transpose_body(tt_ref, tp_ref):
    blk = jnp.swapaxes(tt_ref[0], 0, 1)
    tp_ref[0] = jnp.concatenate([blk[:TLANE // 2], blk[TLANE // 2:]], axis=1)


def _transpose(tt):
    return pl.pallas_call(
        _transpose_body,
        grid=(N_FIELDS, TGRID),
        in_specs=[pl.BlockSpec((1, DIM, TLANE), lambda i, c: (i, 0, c))],
        out_specs=pl.BlockSpec((1, TLANE // 2, 2 * DIM),
                               lambda i, c: (i, c, 0)),
        out_shape=jax.ShapeDtypeStruct(
            (N_FIELDS, TGRID * TLANE // 2, 2 * DIM), jnp.float32),
        compiler_params=pltpu.CompilerParams(
            dimension_semantics=("arbitrary", "arbitrary")),
    )(tt)


def _embed_body(xflat, tablep, out, idx_v, q_v, h_v, rows_v, ob_v, gsem,
                wsem):
    wid = lax.axis_index("s") * NC + lax.axis_index("c")
    pltpu.sync_copy(xflat.at[wid], idx_v)

    def issue(j):
        p = j & 1
        fld = (wid * NCH + j) // CH_PER_F
        off = fld * (TGRID * TLANE // 2)
        for t in range(CHUNK // LANES):
            sl = pl.ds(t * LANES, LANES)
            r = idx_v[j, sl]
            q_v[p, sl] = (lax.shift_right_logical(r, TLANE.bit_length() - 1)
                          * (TLANE // 2) + (r & (TLANE // 2 - 1)) + off)
            h_v[p, sl] = (lax.shift_right_logical(r, TLANE.bit_length() - 2)
                          & 1) * DIM
        pltpu.async_copy(tablep.at[q_v.at[p]], rows_v.at[p], gsem)

    issue(0)

    @pl.loop(0, NCH)
    def _chunk(j):
        p = j & 1

        @pl.when(j + 1 < NCH)
        def _():
            issue(j + 1)

        pltpu.make_async_copy(
            tablep.at[q_v.at[p]], rows_v.at[p], gsem).wait()

        @pl.when(j >= 2)
        def _():
            pltpu.make_async_copy(
                ob_v.at[p], out.at[0, :, pl.ds(0, CHUNK)], wsem).wait()

        for g in range(CHUNK // LANES):
            bvec = lax.iota(jnp.int32, LANES) + g * LANES
            hs = h_v[p, pl.ds(g * LANES, LANES)]

            @pl.loop(0, DIM, unroll=8)
            def _col(c):
                vals = plsc.load_gather(rows_v.at[p], [bvec, hs + c])
                ob_v[p, c, pl.ds(g * LANES, LANES)] = vals

        ch = wid * NCH + j
        fld = ch // CH_PER_F
        b0 = (ch % CH_PER_F) * CHUNK
        pltpu.async_copy(ob_v.at[p], out.at[fld, :, pl.ds(b0, CHUNK)], wsem)

    for k in range(2):
        pltpu.make_async_copy(
            ob_v.at[k], out.at[0, :, pl.ds(0, CHUNK)], wsem).wait()


@functools.partial(
    pl.kernel,
    out_type=jax.ShapeDtypeStruct((N_FIELDS, DIM, BATCH), jnp.float32),
    mesh=plsc.VectorSubcoreMesh(core_axis_name="c", subcore_axis_name="s"),
    compiler_params=pltpu.CompilerParams(needs_layout_passes=False),
    scratch_types=[
        pltpu.VMEM((NCH, CHUNK), jnp.int32),
        pltpu.VMEM((2, CHUNK), jnp.int32),
        pltpu.VMEM((2, CHUNK), jnp.int32),
        pltpu.VMEM((2, CHUNK, 2 * DIM), jnp.float32),
        pltpu.VMEM((2, DIM, CHUNK), jnp.float32),
        pltpu.SemaphoreType.DMA,
        pltpu.SemaphoreType.DMA,
    ],
)
def _embed(xflat, tablep, out, idx_v, q_v, h_v, rows_v, ob_v, gsem, wsem):
    _embed_body(xflat, tablep, out, idx_v, q_v, h_v, rows_v, ob_v, gsem, wsem)


def kernel(x, tables):
    tt = jnp.swapaxes(tables, 1, 2)
    xflat = jnp.swapaxes(x, 0, 1).reshape(NW, NCH, CHUNK)
    tp = _transpose(tt)
    tablep = tp.reshape(N_FIELDS * (TGRID * TLANE // 2), 2 * DIM)
    otv = _embed(xflat, tablep)
    return jnp.transpose(otv, (2, 0, 1))
